# baseline (speedup 1.0000x reference)
import jax
import jax.numpy as jnp
from jax.experimental import pallas as pl

_H = 256


def _lin_kernel(x_ref, w_ref, b_ref, o_ref):
    o_ref[...] = (
        jnp.dot(x_ref[...], w_ref[...], preferred_element_type=jnp.float32)
        + b_ref[...]
    )


def _plin(x, W, b, block=1000):
    N, din = x.shape
    dout = W.shape[0]
    assert N % block == 0, (N, block)
    Wt = W.T
    return pl.pallas_call(
        _lin_kernel,
        grid=(N // block,),
        in_specs=[
            pl.BlockSpec((block, din), lambda i: (i, 0)),
            pl.BlockSpec((din, dout), lambda i: (0, 0)),
            pl.BlockSpec((dout,), lambda i: (0,)),
        ],
        out_specs=pl.BlockSpec((block, dout), lambda i: (i, 0)),
        out_shape=jax.ShapeDtypeStruct((N, dout), jnp.float32),
    )(x, Wt, b)


def _res_kernel(x_ref, w1_ref, b1_ref, w2_ref, b2_ref, o_ref):
    h = (
        jnp.dot(x_ref[...], w1_ref[...], preferred_element_type=jnp.float32)
        + b1_ref[...]
    )
    o_ref[...] = (
        jnp.dot(h, w2_ref[...], preferred_element_type=jnp.float32)
        + b2_ref[...]
        + x_ref[...]
    )


def _pres(p, x, block=1000):
    N, d = x.shape
    W1, b1 = p["l1"]
    W2, b2 = p["l2"]
    return pl.pallas_call(
        _res_kernel,
        grid=(N // block,),
        in_specs=[
            pl.BlockSpec((block, d), lambda i: (i, 0)),
            pl.BlockSpec((d, d), lambda i: (0, 0)),
            pl.BlockSpec((d,), lambda i: (0,)),
            pl.BlockSpec((d, d), lambda i: (0, 0)),
            pl.BlockSpec((d,), lambda i: (0,)),
        ],
        out_specs=pl.BlockSpec((block, d), lambda i: (i, 0)),
        out_shape=jax.ShapeDtypeStruct((N, d), jnp.float32),
    )(x, W1.T, b1, W2.T, b2)


def _edge_kernel(a_ref, b_ref, wk_ref, wf2t_ref, bk_ref, bf2_ref,
                 g_ref, k_ref, f2_ref):
    u = a_ref[...] + b_ref[...]
    h = jnp.where(u >= 0.0, u, 0.2 * u)
    logit = jnp.sum(h * wk_ref[...], axis=1, keepdims=True) + bk_ref[...]
    k = jax.nn.sigmoid(logit)
    g_ref[...] = h * k
    k_ref[...] = k
    t = jnp.dot(h, wf2t_ref[...], preferred_element_type=jnp.float32)
    f2_ref[...] = (t + bf2_ref[...]) * k


def _pedge(Ag, Bg, wk, bk, Wf2, bf2, block=2000):
    E, d2 = Ag.shape
    dout = Wf2.shape[0]
    return pl.pallas_call(
        _edge_kernel,
        grid=(E // block,),
        in_specs=[
            pl.BlockSpec((block, d2), lambda i: (i, 0)),
            pl.BlockSpec((block, d2), lambda i: (i, 0)),
            pl.BlockSpec((1, d2), lambda i: (0, 0)),
            pl.BlockSpec((d2, dout), lambda i: (0, 0)),
            pl.BlockSpec((1, 1), lambda i: (0, 0)),
            pl.BlockSpec((dout,), lambda i: (0,)),
        ],
        out_specs=[
            pl.BlockSpec((block, d2), lambda i: (i, 0)),
            pl.BlockSpec((block, 1), lambda i: (i, 0)),
            pl.BlockSpec((block, dout), lambda i: (i, 0)),
        ],
        out_shape=[
            jax.ShapeDtypeStruct((E, d2), jnp.float32),
            jax.ShapeDtypeStruct((E, 1), jnp.float32),
            jax.ShapeDtypeStruct((E, dout), jnp.float32),
        ],
    )(Ag, Bg, wk.reshape(1, d2), Wf2.T, bk.reshape(1, 1), bf2)


def _mp_direction(x_src, x_dst, edge, msg, red, G, postCat, x_in1, n_dst):
    H = _H
    W1, b1 = msg["l1"]
    W2, b2 = msg["l2"]
    A = _plin(x_src, W1[:, :H], jnp.zeros((2 * H,), jnp.float32))
    B = _plin(x_dst, W1[:, H:], b1)
    Ag = A[edge[0]]
    Bg = B[edge[1]]
    wk = W2[0]
    bk = b2[0:1]
    Wf1 = W2[1 : 1 + H]
    bf1 = b2[1 : 1 + H]
    Wf2 = W2[1 + H :]
    bf2 = b2[1 + H :]
    g, k, f2 = _pedge(Ag, Bg, wk, bk, Wf2, bf2)
    S = jax.ops.segment_sum(g, edge[1], num_segments=n_dst)
    ksum = jax.ops.segment_sum(k[:, 0], edge[1], num_segments=n_dst)
    nf1 = _plin(S, Wf1, jnp.zeros((H,), jnp.float32)) + bf1 * ksum[:, None]
    m = jax.ops.segment_max(f2, edge[1], num_segments=n_dst)
    nf2 = jnp.where(jnp.isneginf(m), 0.0, m)
    cat = jnp.concatenate([x_dst, nf1, nf2], axis=1)
    new_x = _plin(cat, red[0], red[1])
    new_x = _plin(new_x, G[0], G[1])
    cat2 = jnp.concatenate([new_x, x_in1], axis=1)
    return x_dst + _plin(cat2, postCat[0], postCat[1])


def kernel(nf_gc, nf_gn, nf_gc_in1, nf_gn_in1, edge_c2n, edge_n2c, params):
    p = params
    x_gc_in1 = _plin(nf_gc_in1, p["gc_in1"][0], p["gc_in1"][1])
    x_gn_in1 = _plin(nf_gn_in1, p["gn_in1"][0], p["gn_in1"][1])
    x_gc = _pres(p["res_gc_1"], nf_gc)
    x_gn = _pres(p["res_gn_1"], nf_gn)
    NN = nf_gn.shape[0]
    NC = nf_gc.shape[0]
    x_gn = _mp_direction(
        x_gc, x_gn, edge_c2n, p["msg_c2n"], p["red_c2n"], p["Gcn"],
        p["postCatGcn"], x_gn_in1, NN,
    )
    x_gn = _pres(p["res_gn_2"], x_gn)
    x_gc = _pres(p["res_gc_2"], x_gc)
    x_gc = _mp_direction(
        x_gn, x_gc, edge_n2c, p["msg_n2c"], p["red_n2c"], p["Gnc"],
        p["postCatGnc"], x_gc_in1, NC,
    )
    return (x_gc, x_gn)


# speedup vs baseline: 1.3296x; 1.3296x over previous
import jax
import jax.numpy as jnp
from jax.experimental import pallas as pl

_H = 256


def _lin_kernel(x_ref, w_ref, b_ref, o_ref):
    o_ref[...] = (
        jnp.dot(x_ref[...], w_ref[...], preferred_element_type=jnp.float32)
        + b_ref[...]
    )


def _plin(x, W, b, block=1000):
    N, din = x.shape
    dout = W.shape[0]
    assert N % block == 0, (N, block)
    Wt = W.T
    return pl.pallas_call(
        _lin_kernel,
        grid=(N // block,),
        in_specs=[
            pl.BlockSpec((block, din), lambda i: (i, 0)),
            pl.BlockSpec((din, dout), lambda i: (0, 0)),
            pl.BlockSpec((dout,), lambda i: (0,)),
        ],
        out_specs=pl.BlockSpec((block, dout), lambda i: (i, 0)),
        out_shape=jax.ShapeDtypeStruct((N, dout), jnp.float32),
    )(x, Wt, b)


def _res_kernel(x_ref, w1_ref, b1_ref, w2_ref, b2_ref, o_ref):
    h = (
        jnp.dot(x_ref[...], w1_ref[...], preferred_element_type=jnp.float32)
        + b1_ref[...]
    )
    o_ref[...] = (
        jnp.dot(h, w2_ref[...], preferred_element_type=jnp.float32)
        + b2_ref[...]
        + x_ref[...]
    )


def _pres(p, x, block=1000):
    N, d = x.shape
    W1, b1 = p["l1"]
    W2, b2 = p["l2"]
    return pl.pallas_call(
        _res_kernel,
        grid=(N // block,),
        in_specs=[
            pl.BlockSpec((block, d), lambda i: (i, 0)),
            pl.BlockSpec((d, d), lambda i: (0, 0)),
            pl.BlockSpec((d,), lambda i: (0,)),
            pl.BlockSpec((d, d), lambda i: (0, 0)),
            pl.BlockSpec((d,), lambda i: (0,)),
        ],
        out_specs=pl.BlockSpec((block, d), lambda i: (i, 0)),
        out_shape=jax.ShapeDtypeStruct((N, d), jnp.float32),
    )(x, W1.T, b1, W2.T, b2)


def _edge_kernel(a_ref, b_ref, wk_ref, w2t_ref, bk_ref, b2_ref,
                 f1_ref, f2_ref):
    H = _H
    u = a_ref[...].astype(jnp.float32) + b_ref[...].astype(jnp.float32)
    h = jnp.where(u >= 0.0, u, 0.2 * u)
    logit = jnp.sum(h * wk_ref[...], axis=1, keepdims=True) + bk_ref[...]
    k = jax.nn.sigmoid(logit)
    m2 = (
        jnp.dot(h.astype(jnp.bfloat16), w2t_ref[...],
                preferred_element_type=jnp.float32)
        + b2_ref[...]
    )
    f = m2 * k
    f1_ref[...] = f[:, :H]
    f2_ref[...] = f[:, H:]


def _pedge(Ag, Bg, wk, bk, W2r, b2r, block=2000):
    E, d2 = Ag.shape
    H = _H
    return pl.pallas_call(
        _edge_kernel,
        grid=(E // block,),
        in_specs=[
            pl.BlockSpec((block, d2), lambda i: (i, 0)),
            pl.BlockSpec((block, d2), lambda i: (i, 0)),
            pl.BlockSpec((1, d2), lambda i: (0, 0)),
            pl.BlockSpec((d2, d2), lambda i: (0, 0)),
            pl.BlockSpec((1, 1), lambda i: (0, 0)),
            pl.BlockSpec((d2,), lambda i: (0,)),
        ],
        out_specs=[
            pl.BlockSpec((block, H), lambda i: (i, 0)),
            pl.BlockSpec((block, H), lambda i: (i, 0)),
        ],
        out_shape=[
            jax.ShapeDtypeStruct((E, H), jnp.float32),
            jax.ShapeDtypeStruct((E, H), jnp.float32),
        ],
    )(Ag, Bg, wk.reshape(1, d2), W2r.T.astype(jnp.bfloat16),
      bk.reshape(1, 1), b2r)


def _mp_direction(x_src, x_dst, edge, msg, red, G, postCat, x_in1, n_dst):
    H = _H
    W1, b1 = msg["l1"]
    W2, b2 = msg["l2"]
    A = _plin(x_src, W1[:, :H], jnp.zeros((2 * H,), jnp.float32))
    B = _plin(x_dst, W1[:, H:], b1)
    Ag = A.astype(jnp.bfloat16)[edge[0]]
    Bg = B.astype(jnp.bfloat16)[edge[1]]
    wk = W2[0]
    bk = b2[0:1]
    f1, f2 = _pedge(Ag, Bg, wk, bk, W2[1:], b2[1:])
    nf1 = jax.ops.segment_sum(f1, edge[1], num_segments=n_dst)
    m = jax.ops.segment_max(f2, edge[1], num_segments=n_dst)
    nf2 = jnp.where(jnp.isneginf(m), 0.0, m)
    cat = jnp.concatenate([x_dst, nf1, nf2], axis=1)
    new_x = _plin(cat, red[0], red[1])
    new_x = _plin(new_x, G[0], G[1])
    cat2 = jnp.concatenate([new_x, x_in1], axis=1)
    return x_dst + _plin(cat2, postCat[0], postCat[1])


def kernel(nf_gc, nf_gn, nf_gc_in1, nf_gn_in1, edge_c2n, edge_n2c, params):
    p = params
    x_gc_in1 = _plin(nf_gc_in1, p["gc_in1"][0], p["gc_in1"][1])
    x_gn_in1 = _plin(nf_gn_in1, p["gn_in1"][0], p["gn_in1"][1])
    x_gc = _pres(p["res_gc_1"], nf_gc)
    x_gn = _pres(p["res_gn_1"], nf_gn)
    NN = nf_gn.shape[0]
    NC = nf_gc.shape[0]
    x_gn = _mp_direction(
        x_gc, x_gn, edge_c2n, p["msg_c2n"], p["red_c2n"], p["Gcn"],
        p["postCatGcn"], x_gn_in1, NN,
    )
    x_gn = _pres(p["res_gn_2"], x_gn)
    x_gc = _pres(p["res_gc_2"], x_gc)
    x_gc = _mp_direction(
        x_gn, x_gc, edge_n2c, p["msg_n2c"], p["red_n2c"], p["Gnc"],
        p["postCatGnc"], x_gc_in1, NC,
    )
    return (x_gc, x_gn)
